# baseline (speedup 1.0000x reference)
import functools

import jax
import jax.numpy as jnp
from jax import lax
from jax.experimental import pallas as pl
from jax.experimental.pallas import tpu as pltpu
from jax.experimental.pallas import tpu_sc as plsc

EMBED_DIM = 64
BATCH = 16384
NUM_TABLES = 8

_info = plsc.get_sparse_core_info()
_NC, _NS = _info.num_cores, _info.num_subcores
_NW = _NC * _NS
_BPW = BATCH // _NW
_CHUNK = 128
_NCHUNK = _BPW // _CHUNK


def _body(*refs):
    tables = refs[0:NUM_TABLES]
    idxs = refs[NUM_TABLES:2 * NUM_TABLES]
    outs = refs[2 * NUM_TABLES:3 * NUM_TABLES]
    idx_v, rows_v, sem = refs[3 * NUM_TABLES:]

    wid = lax.axis_index("s") * _NC + lax.axis_index("c")
    base = wid * _BPW
    for t in range(NUM_TABLES):
        pltpu.sync_copy(idxs[t].at[pl.ds(base, _BPW)], idx_v)

        def jbody(j, carry, t=t):
            pltpu.async_copy(
                tables[t].at[idx_v.at[pl.ds(j * _CHUNK, _CHUNK)]],
                rows_v, sem).wait()
            pltpu.sync_copy(rows_v, outs[t].at[pl.ds(base + j * _CHUNK, _CHUNK)])
            return carry
        lax.fori_loop(0, _NCHUNK, jbody, 0)


@jax.jit
def kernel(patient_emb, visit_emb, symptom_emb, procedure_emb, disease_emb,
           drug_emb, anatomy_emb, pharmaclass_emb,
           patient_node_id, visit_node_id, symptom_node_id, procedure_node_id,
           disease_node_id, drug_node_id, anatomy_node_id, pharmaclass_node_id):
    out_type = tuple(
        jax.ShapeDtypeStruct((BATCH, EMBED_DIM), jnp.float32)
        for _ in range(NUM_TABLES)
    )
    k = functools.partial(
        pl.kernel,
        mesh=plsc.VectorSubcoreMesh(core_axis_name="c", subcore_axis_name="s"),
        out_type=out_type,
        scratch_types=[
            pltpu.VMEM((_BPW,), jnp.int32),
            pltpu.VMEM((_CHUNK, EMBED_DIM), jnp.float32),
            pltpu.SemaphoreType.DMA,
        ],
        compiler_params=pltpu.CompilerParams(use_tc_tiling_on_sc=False),
    )(_body)
    return k(patient_emb, visit_emb, symptom_emb, procedure_emb, disease_emb,
             drug_emb, anatomy_emb, pharmaclass_emb,
             patient_node_id, visit_node_id, symptom_node_id,
             procedure_node_id, disease_node_id, drug_node_id,
             anatomy_node_id, pharmaclass_node_id)


# speedup vs baseline: 1.5101x; 1.5101x over previous
import functools

import jax
import jax.numpy as jnp
from jax import lax
from jax.experimental import pallas as pl
from jax.experimental.pallas import tpu as pltpu
from jax.experimental.pallas import tpu_sc as plsc

EMBED_DIM = 64
BATCH = 16384
NUM_TABLES = 8

_info = plsc.get_sparse_core_info()
_NC, _NS = _info.num_cores, _info.num_subcores
_NW = _NC * _NS
_BPW = BATCH // _NW


def _body(*refs):
    tables = refs[0:NUM_TABLES]
    idxs = refs[NUM_TABLES:2 * NUM_TABLES]
    outs = refs[2 * NUM_TABLES:3 * NUM_TABLES]
    idx_v, row_v, sem = refs[3 * NUM_TABLES:]

    wid = lax.axis_index("s") * _NC + lax.axis_index("c")
    base = wid * _BPW
    for t in range(NUM_TABLES):
        pltpu.sync_copy(idxs[t].at[pl.ds(base, _BPW)], idx_v)

        def fire(g, carry, t=t):
            vec = idx_v[pl.ds(g * 16, 16)]
            for j in range(16):
                pltpu.async_copy(tables[t].at[vec[j]],
                                 row_v.at[g * 16 + j], sem)
            return carry
        lax.fori_loop(0, _BPW // 16, fire, 0)

        def drain(i, carry, t=t):
            pltpu.make_async_copy(tables[t].at[0], row_v.at[i], sem).wait()
            return carry
        lax.fori_loop(0, _BPW, drain, 0)
        pltpu.sync_copy(row_v, outs[t].at[pl.ds(base, _BPW)])


@jax.jit
def kernel(patient_emb, visit_emb, symptom_emb, procedure_emb, disease_emb,
           drug_emb, anatomy_emb, pharmaclass_emb,
           patient_node_id, visit_node_id, symptom_node_id, procedure_node_id,
           disease_node_id, drug_node_id, anatomy_node_id, pharmaclass_node_id):
    out_type = tuple(
        jax.ShapeDtypeStruct((BATCH, EMBED_DIM), jnp.float32)
        for _ in range(NUM_TABLES)
    )
    k = functools.partial(
        pl.kernel,
        mesh=plsc.VectorSubcoreMesh(core_axis_name="c", subcore_axis_name="s"),
        out_type=out_type,
        scratch_types=[
            pltpu.VMEM((_BPW,), jnp.int32),
            pltpu.VMEM((_BPW, EMBED_DIM), jnp.float32),
            pltpu.SemaphoreType.DMA,
        ],
        compiler_params=pltpu.CompilerParams(needs_layout_passes=False),
    )(_body)
    return k(patient_emb, visit_emb, symptom_emb, procedure_emb, disease_emb,
             drug_emb, anatomy_emb, pharmaclass_emb,
             patient_node_id, visit_node_id, symptom_node_id,
             procedure_node_id, disease_node_id, drug_node_id,
             anatomy_node_id, pharmaclass_node_id)
